# Initial kernel scaffold; baseline (speedup 1.0000x reference)
import jax
import jax.numpy as jnp
from jax.experimental import pallas as pl


def kernel(x, bn1_gamma, bn1_beta, gate1_W, gate1_b, exp1_W, exp1_b, bn2_gamma, bn2_beta, gate2_W, gate2_b, exp2_W, exp2_b, out_W, out_b):
    raise NotImplementedError("write your pallas kernel here")


# speedup vs baseline: 1.9216x; 1.9216x over previous
import jax
import jax.numpy as jnp
from jax.experimental import pallas as pl
from jax.experimental.pallas import tpu as pltpu

N = 2048
D = 768
H = 768
O = 768
E = 8
K = 2
HH = H // 2
CH = 256


def _dotd(a, b):
    return jnp.dot(a.astype(jnp.bfloat16), b.astype(jnp.bfloat16),
                   preferred_element_type=jnp.float32)


def _bn(x, gamma, beta, eps=1e-5):
    mu = jnp.mean(x, axis=0, keepdims=True)
    var = jnp.mean((x - mu) ** 2, axis=0, keepdims=True)
    return (x - mu) / jnp.sqrt(var + eps) * gamma + beta


def _top2_combine(logits):
    it = jax.lax.broadcasted_iota(jnp.int32, logits.shape, 1)
    v1 = jnp.max(logits, axis=1, keepdims=True)
    i1 = jnp.min(jnp.where(logits == v1, it, E), axis=1, keepdims=True)
    m1 = it == i1
    masked = jnp.where(m1, -jnp.inf, logits)
    v2 = jnp.max(masked, axis=1, keepdims=True)
    i2 = jnp.min(jnp.where(masked == v2, it, E), axis=1, keepdims=True)
    t = jnp.exp(v2 - v1)
    z = 1.0 + t
    g1 = 1.0 / z
    g2 = t / z
    return g1 * m1.astype(logits.dtype) + g2 * (it == i2).astype(logits.dtype)


def _expert_accum(e, xn_s, c_s, W, bvec, out_ref):
    def body(i, _):
        sl = pl.ds(i * CH, CH)
        xc = xn_s[sl, :]
        part = _dotd(xc, W) + bvec
        it = jax.lax.broadcasted_iota(jnp.int32, (CH, E), 1)
        w = jnp.sum(c_s[sl, :] * (it == e).astype(jnp.float32),
                    axis=1, keepdims=True)
        w16 = w.astype(jnp.bfloat16).astype(jnp.float32)
        p16 = part.astype(jnp.bfloat16).astype(jnp.float32)
        contrib = w16 * p16
        prev = out_ref[sl, :]
        out_ref[sl, :] = jnp.where(e == 0, contrib, prev + contrib)
        return 0

    jax.lax.fori_loop(0, N // CH, body, 0)


def _stage1_kernel(x_ref, bn1g, bn1b, g1W, g1b, W_ref, b_ref, bn2g, bn2b,
                   g2W, g2b, hn_out, c2_out, xn_s, c1_s):
    e = pl.program_id(0)

    @pl.when(e == 0)
    def _():
        xn = _bn(x_ref[...], bn1g[...], bn1b[...])
        xn_s[...] = xn
        logits = _dotd(xn, g1W[...])
        c1_s[...] = _top2_combine(logits + g1b[...])

    _expert_accum(e, xn_s, c1_s, W_ref[0], b_ref[0], hn_out)

    @pl.when(e == E - 1)
    def _():
        h = hn_out[...]
        z = jnp.maximum(_bn(h, bn2g[...], bn2b[...]), 0.0)
        hn_out[...] = z
        logits2 = _dotd(z, g2W[...])
        c2_out[...] = _top2_combine(logits2 + g2b[...])


def _stage2_kernel(hn_ref, c2_ref, W_ref, b_ref, oW, ob, y_out, h2_s):
    e = pl.program_id(0)
    _expert_accum(e, hn_ref, c2_ref, W_ref[0], b_ref[0], h2_s)

    @pl.when(e == E - 1)
    def _():
        def body(i, _):
            sl = pl.ds(i * CH, CH)
            z = jnp.maximum(h2_s[sl, :], 0.0)
            y_out[sl, :] = _dotd(z, oW[...]) + ob[...]
            return 0

        jax.lax.fori_loop(0, N // CH, body, 0)


def kernel(x, bn1_gamma, bn1_beta, gate1_W, gate1_b, exp1_W, exp1_b,
           bn2_gamma, bn2_beta, gate2_W, gate2_b, exp2_W, exp2_b, out_W, out_b):
    bn1g = bn1_gamma.reshape(1, D)
    bn1b = bn1_beta.reshape(1, D)
    g1b = gate1_b.reshape(1, E)
    bn2g = bn2_gamma.reshape(1, H)
    bn2b = bn2_beta.reshape(1, H)
    g2b = gate2_b.reshape(1, E)
    ob = out_b.reshape(1, O)

    const = lambda *blk: pl.BlockSpec(blk, lambda e: (0,) * len(blk))

    hn, c2 = pl.pallas_call(
        _stage1_kernel,
        grid=(E,),
        in_specs=[
            const(N, D),
            const(1, D), const(1, D),
            const(D, E), const(1, E),
            pl.BlockSpec((1, D, H), lambda e: (e, 0, 0)),
            pl.BlockSpec((1, 1, H), lambda e: (e, 0, 0)),
            const(1, H), const(1, H),
            const(H, E), const(1, E),
        ],
        out_specs=[const(N, H), const(N, E)],
        out_shape=[
            jax.ShapeDtypeStruct((N, H), jnp.float32),
            jax.ShapeDtypeStruct((N, E), jnp.float32),
        ],
        scratch_shapes=[
            pltpu.VMEM((N, D), jnp.float32),
            pltpu.VMEM((N, E), jnp.float32),
        ],
        compiler_params=pltpu.CompilerParams(
            dimension_semantics=("arbitrary",)),
    )(x, bn1g, bn1b, gate1_W, g1b, exp1_W, exp1_b.reshape(E, 1, H),
      bn2g, bn2b, gate2_W, g2b)

    y = pl.pallas_call(
        _stage2_kernel,
        grid=(E,),
        in_specs=[
            const(N, H),
            const(N, E),
            pl.BlockSpec((1, H, HH), lambda e: (e, 0, 0)),
            pl.BlockSpec((1, 1, HH), lambda e: (e, 0, 0)),
            const(HH, O), const(1, O),
        ],
        out_specs=const(N, O),
        out_shape=jax.ShapeDtypeStruct((N, O), jnp.float32),
        scratch_shapes=[pltpu.VMEM((N, HH), jnp.float32)],
        compiler_params=pltpu.CompilerParams(
            dimension_semantics=("arbitrary",)),
    )(hn, c2, exp2_W, exp2_b.reshape(E, 1, HH), out_W, ob)

    return y


# speedup vs baseline: 2.1520x; 1.1199x over previous
import jax
import jax.numpy as jnp
from jax.experimental import pallas as pl
from jax.experimental.pallas import tpu as pltpu

N = 2048
D = 768
H = 768
O = 768
E = 8
K = 2
HH = H // 2
TM = 128
NT = N // TM


def _bn(x, gamma, beta, eps=1e-5):
    mu = jnp.mean(x, axis=0, keepdims=True)
    var = jnp.mean((x - mu) ** 2, axis=0, keepdims=True)
    return (x - mu) / jnp.sqrt(var + eps) * gamma + beta


def _round16(x):
    return x.astype(jnp.bfloat16).astype(jnp.float32)


def _top2_combine(logits):
    it = jax.lax.broadcasted_iota(jnp.int32, logits.shape, 1)
    v1 = jnp.max(logits, axis=1, keepdims=True)
    i1 = jnp.min(jnp.where(logits == v1, it, E), axis=1, keepdims=True)
    m1 = it == i1
    masked = jnp.where(m1, -jnp.inf, logits)
    v2 = jnp.max(masked, axis=1, keepdims=True)
    i2 = jnp.min(jnp.where(masked == v2, it, E), axis=1, keepdims=True)
    t = jnp.exp(v2 - v1)
    z = 1.0 + t
    g1 = 1.0 / z
    g2 = t / z
    return g1 * m1.astype(logits.dtype) + g2 * (it == i2).astype(logits.dtype)


def _gate1_kernel(x_ref, bn1g, bn1b, gW, gb, xn16_out, c_out):
    xn = _bn(x_ref[...], bn1g[...], bn1b[...])
    x16 = xn.astype(jnp.bfloat16)
    xn16_out[...] = x16
    logits = jnp.dot(x16, gW[...], preferred_element_type=jnp.float32)
    c_out[...] = _round16(_top2_combine(logits + gb[...]))


def _gate2_kernel(h_ref, bn2g, bn2b, gW, gb, zn16_out, c_out):
    z = jnp.maximum(_bn(h_ref[...], bn2g[...], bn2b[...]), 0.0)
    z16 = z.astype(jnp.bfloat16)
    zn16_out[...] = z16
    logits = jnp.dot(z16, gW[...], preferred_element_type=jnp.float32)
    c_out[...] = _round16(_top2_combine(logits + gb[...]))


def _moe1_kernel(x16_ref, c_ref, W_ref, b_ref, h_out):
    x16 = x16_ref[...]
    acc = None
    for e in range(E):
        part = jnp.dot(x16, W_ref[e], preferred_element_type=jnp.float32)
        p16 = _round16(part + b_ref[e])
        contrib = c_ref[:, e:e + 1] * p16
        acc = contrib if acc is None else acc + contrib
    h_out[...] = acc


def _moe2_out_kernel(z16_ref, c_ref, W_ref, b_ref, oW, ob, y_out):
    z16 = z16_ref[...]
    acc = None
    for e in range(E):
        part = jnp.dot(z16, W_ref[e], preferred_element_type=jnp.float32)
        p16 = _round16(part + b_ref[e])
        contrib = c_ref[:, e:e + 1] * p16
        acc = contrib if acc is None else acc + contrib
    r16 = jnp.maximum(acc, 0.0).astype(jnp.bfloat16)
    y_out[...] = jnp.dot(r16, oW[...], preferred_element_type=jnp.float32) + ob[...]


def kernel(x, bn1_gamma, bn1_beta, gate1_W, gate1_b, exp1_W, exp1_b,
           bn2_gamma, bn2_beta, gate2_W, gate2_b, exp2_W, exp2_b, out_W, out_b):
    bn1g = bn1_gamma.reshape(1, D)
    bn1b = bn1_beta.reshape(1, D)
    g1b = gate1_b.reshape(1, E)
    bn2g = bn2_gamma.reshape(1, H)
    bn2b = bn2_beta.reshape(1, H)
    g2b = gate2_b.reshape(1, E)
    ob = out_b.reshape(1, O)

    g1W16 = gate1_W.astype(jnp.bfloat16)
    g2W16 = gate2_W.astype(jnp.bfloat16)
    e1W16 = exp1_W.astype(jnp.bfloat16)
    e2W16 = exp2_W.astype(jnp.bfloat16)
    oW16 = out_W.astype(jnp.bfloat16)
    e1b = exp1_b.reshape(E, 1, H)
    e2b = exp2_b.reshape(E, 1, HH)

    whole = lambda *blk: pl.BlockSpec(blk, lambda *_: (0,) * len(blk))

    xn16, c1 = pl.pallas_call(
        _gate1_kernel,
        in_specs=[whole(N, D), whole(1, D), whole(1, D),
                  whole(D, E), whole(1, E)],
        out_specs=[whole(N, D), whole(N, E)],
        out_shape=[jax.ShapeDtypeStruct((N, D), jnp.bfloat16),
                   jax.ShapeDtypeStruct((N, E), jnp.float32)],
    )(x, bn1g, bn1b, g1W16, g1b)

    h = pl.pallas_call(
        _moe1_kernel,
        grid=(NT,),
        in_specs=[
            pl.BlockSpec((TM, D), lambda i: (i, 0)),
            pl.BlockSpec((TM, E), lambda i: (i, 0)),
            pl.BlockSpec((E, D, H), lambda i: (0, 0, 0)),
            pl.BlockSpec((E, 1, H), lambda i: (0, 0, 0)),
        ],
        out_specs=pl.BlockSpec((TM, H), lambda i: (i, 0)),
        out_shape=jax.ShapeDtypeStruct((N, H), jnp.float32),
        compiler_params=pltpu.CompilerParams(
            dimension_semantics=("parallel",)),
    )(xn16, c1, e1W16, e1b)

    zn16, c2 = pl.pallas_call(
        _gate2_kernel,
        in_specs=[whole(N, H), whole(1, H), whole(1, H),
                  whole(H, E), whole(1, E)],
        out_specs=[whole(N, H), whole(N, E)],
        out_shape=[jax.ShapeDtypeStruct((N, H), jnp.bfloat16),
                   jax.ShapeDtypeStruct((N, E), jnp.float32)],
    )(h, bn2g, bn2b, g2W16, g2b)

    y = pl.pallas_call(
        _moe2_out_kernel,
        grid=(NT,),
        in_specs=[
            pl.BlockSpec((TM, H), lambda i: (i, 0)),
            pl.BlockSpec((TM, E), lambda i: (i, 0)),
            pl.BlockSpec((E, H, HH), lambda i: (0, 0, 0)),
            pl.BlockSpec((E, 1, HH), lambda i: (0, 0, 0)),
            whole(HH, O), whole(1, O),
        ],
        out_specs=pl.BlockSpec((TM, O), lambda i: (i, 0)),
        out_shape=jax.ShapeDtypeStruct((N, O), jnp.float32),
        compiler_params=pltpu.CompilerParams(
            dimension_semantics=("parallel",)),
    )(zn16, c2, e2W16, e2b, oW16, ob)

    return y
